# baseline (speedup 1.0000x reference)
import functools

import jax
import jax.numpy as jnp
from jax import lax
from jax.experimental import pallas as pl
from jax.experimental.pallas import tpu as pltpu
from jax.experimental.pallas import tpu_sc as plsc

_NS = 16
_C = 80


def _sc_segment_sums(N, H, E):
    EPT = E // _NS
    CHUNKS = EPT // _C
    RPT = N // 10
    ND = ((N + 1279) // 1280) * 1280
    DCH = ND // 10
    assert EPT * _NS == E and CHUNKS * _C == EPT and RPT * 10 == N
    assert RPT % 8 == 0 and DCH % 128 == 0 and EPT % 8 == 0 and _C % 8 == 0

    mesh = plsc.VectorSubcoreMesh(core_axis_name="c", subcore_axis_name="s")

    NB = 4
    GROUPS = CHUNKS // NB
    TAIL = CHUNKS - GROUPS * NB

    def body(x_hbm, src_hbm, dst_hbm, ea_hbm, zrows_hbm, zdeg_hbm,
             a_hbm, b_hbm, deg_hbm,
             acc, degacc, idx_s, idx_d, rows_v, ones_v,
             isem, gsem, ssem):
        c = lax.axis_index("c")
        s = lax.axis_index("s")

        @pl.when(s < 10)
        def _zero():
            pltpu.sync_copy(zrows_hbm, acc.at[pl.ds(s * RPT, RPT), :])
            pltpu.sync_copy(zdeg_hbm, degacc.at[pl.ds(s * DCH, DCH)])

        for k in range(_C // 16):
            ones_v[pl.ds(k * 16, 16)] = jnp.full((16,), 1.0, jnp.float32)

        plsc.subcore_barrier()

        tbase = s * EPT
        HALF = NB // 2
        HA = list(range(HALF))
        HB = list(range(HALF, NB))

        def _fire0(slots, goff):
            for i, b in enumerate(slots):
                off = pl.multiple_of(tbase + goff + i * _C, 8)
                pltpu.async_copy(src_hbm.at[pl.ds(off, _C)], idx_s.at[b],
                                 isem.at[b])
                pltpu.async_copy(dst_hbm.at[pl.ds(off, _C)], idx_d.at[b],
                                 isem.at[b])

        def _fire1(slots, goff):
            for i, b in enumerate(slots):
                off = pl.multiple_of(tbase + goff + i * _C, 8)
                pltpu.async_copy(dst_hbm.at[pl.ds(off, _C)], idx_d.at[b],
                                 isem.at[b])
                pltpu.async_copy(ea_hbm.at[pl.ds(off, _C), :], rows_v.at[b],
                                 gsem.at[b])

        def _wait_scat(slots, with_deg):
            for b in slots:
                pltpu.make_async_copy(rows_v.at[b], acc.at[idx_d.at[b]],
                                      ssem.at[b]).wait()
                if with_deg:
                    pltpu.make_async_copy(ones_v, degacc.at[idx_d.at[b]],
                                          ssem.at[b]).wait()

        def _proc1(slots):
            for b in slots:
                pltpu.make_async_copy(dst_hbm.at[pl.ds(0, _C)],
                                      idx_d.at[b], isem.at[b]).wait()
                pltpu.make_async_copy(ea_hbm.at[pl.ds(0, _C), :],
                                      rows_v.at[b], gsem.at[b]).wait()
                pltpu.async_copy(rows_v.at[b], acc.at[idx_d.at[b]],
                                 ssem.at[b], add=True)
                pltpu.async_copy(ones_v, degacc.at[idx_d.at[b]], ssem.at[b],
                                 add=True)

        def _pipeline(fire, proc, with_deg):
            fire(HA, 0)

            def group(g, carry):
                base = g * (NB * _C)

                @pl.when(g > 0)
                def _():
                    _wait_scat(HB, with_deg)
                fire(HB, base + HALF * _C)
                proc(HA)
                _wait_scat(HA, with_deg)

                @pl.when(g + 1 < GROUPS)
                def _():
                    fire(HA, base + NB * _C)
                proc(HB)
                return carry
            lax.fori_loop(0, GROUPS, group, 0)
            _wait_scat(HB, with_deg)

        def _pipeline0():
            _fire0(HA, 0)

            def group(g, carry):
                base = g * (NB * _C)

                @pl.when(g > 0)
                def _():
                    _wait_scat(HB, False)
                _fire0(HB, base + HALF * _C)
                gd = {}
                for b in HA + HB:
                    pltpu.make_async_copy(src_hbm.at[pl.ds(0, _C)],
                                          idx_s.at[b], isem.at[b]).wait()
                    pltpu.make_async_copy(dst_hbm.at[pl.ds(0, _C)],
                                          idx_d.at[b], isem.at[b]).wait()
                    gd[b] = pltpu.async_copy(x_hbm.at[idx_s.at[b]],
                                             rows_v.at[b], gsem.at[b])
                for b in HA:
                    gd[b].wait()
                    pltpu.async_copy(rows_v.at[b], acc.at[idx_d.at[b]],
                                     ssem.at[b], add=True)
                _wait_scat(HA, False)

                @pl.when(g + 1 < GROUPS)
                def _():
                    _fire0(HA, base + NB * _C)
                for b in HB:
                    gd[b].wait()
                    pltpu.async_copy(rows_v.at[b], acc.at[idx_d.at[b]],
                                     ssem.at[b], add=True)
                return carry
            lax.fori_loop(0, GROUPS, group, 0)
            _wait_scat(HB, False)

        @pl.when(c == 0)
        def _core0():
            _pipeline0()
            for t in range(TAIL):
                off = pl.multiple_of(tbase + (GROUPS * NB + t) * _C, 8)
                pltpu.sync_copy(src_hbm.at[pl.ds(off, _C)], idx_s.at[0])
                pltpu.sync_copy(dst_hbm.at[pl.ds(off, _C)], idx_d.at[0])
                pltpu.async_copy(x_hbm.at[idx_s.at[0]], rows_v.at[0],
                                 gsem.at[0]).wait()
                pltpu.sync_copy(rows_v.at[0], acc.at[idx_d.at[0]], add=True)

        @pl.when(c == 1)
        def _core1():
            _pipeline(_fire1, _proc1, True)
            for t in range(TAIL):
                off = pl.multiple_of(tbase + (GROUPS * NB + t) * _C, 8)
                pltpu.sync_copy(dst_hbm.at[pl.ds(off, _C)], idx_d.at[0])
                pltpu.sync_copy(ea_hbm.at[pl.ds(off, _C), :], rows_v.at[0])
                pltpu.sync_copy(rows_v.at[0], acc.at[idx_d.at[0]], add=True)
                pltpu.sync_copy(ones_v, degacc.at[idx_d.at[0]], add=True)

        plsc.subcore_barrier()

        @pl.when(jnp.logical_and(c == 0, s < 10))
        def _out0():
            pltpu.sync_copy(acc.at[pl.ds(s * RPT, RPT), :],
                            a_hbm.at[pl.ds(s * RPT, RPT), :])

        @pl.when(jnp.logical_and(c == 1, s < 10))
        def _out1():
            pltpu.sync_copy(acc.at[pl.ds(s * RPT, RPT), :],
                            b_hbm.at[pl.ds(s * RPT, RPT), :])
            pltpu.sync_copy(degacc.at[pl.ds(s * DCH, DCH)],
                            deg_hbm.at[pl.ds(s * DCH, DCH)])

    return pl.kernel(
        body,
        out_type=(jax.ShapeDtypeStruct((N, H), jnp.float32),
                  jax.ShapeDtypeStruct((N, H), jnp.float32),
                  jax.ShapeDtypeStruct((ND,), jnp.float32)),
        mesh=mesh,
        scratch_types=[
            pltpu.VMEM_SHARED((N, H), jnp.float32),
            pltpu.VMEM_SHARED((ND,), jnp.float32),
            pltpu.VMEM((NB, _C), jnp.int32),
            pltpu.VMEM((NB, _C), jnp.int32),
            pltpu.VMEM((NB, _C, H), jnp.float32),
            pltpu.VMEM((_C,), jnp.float32),
            pltpu.SemaphoreType.DMA((NB,)),
            pltpu.SemaphoreType.DMA((NB,)),
            pltpu.SemaphoreType.DMA((NB,)),
        ],
    )


def _dot_nt(lhs, w):
    return lax.dot_general(lhs, w, (((1,), (1,)), ((), ())),
                           preferred_element_type=jnp.float32)


_BLK = 2000


def _self_body(x_ref, ws_ref, bias_ref, o_ref):
    o_ref[...] = _dot_nt(x_ref[...], ws_ref[...]) + bias_ref[...]


def _tc_self(N, H, x, ws, bias):
    row_spec = pl.BlockSpec((_BLK, H), lambda i: (i, 0))
    return pl.pallas_call(
        _self_body,
        grid=(N // _BLK,),
        in_specs=[row_spec,
                  pl.BlockSpec((H, H), lambda i: (0, 0)),
                  pl.BlockSpec((1, H), lambda i: (0, 0))],
        out_specs=row_spec,
        out_shape=jax.ShapeDtypeStruct((N, H), jnp.float32),
    )(x, ws, bias)


def _tc_body(p_ref, a_ref, b_ref, deg_ref, wn_ref, we_ref, o_ref):
    agg = _dot_nt(a_ref[...], wn_ref[...]) + _dot_nt(b_ref[...], we_ref[...])
    agg = agg / jnp.maximum(deg_ref[...], 1.0)
    o_ref[...] = jnp.maximum(p_ref[...] + agg, 0.0)


def _tc_combine(N, H, p, a, b, deg, wn, we):
    assert N % _BLK == 0
    row_spec = pl.BlockSpec((_BLK, H), lambda i: (i, 0))
    w_spec = pl.BlockSpec((H, H), lambda i: (0, 0))
    return pl.pallas_call(
        _tc_body,
        grid=(N // _BLK,),
        in_specs=[
            row_spec,
            row_spec,
            row_spec,
            pl.BlockSpec((_BLK, 1), lambda i: (i, 0)),
            w_spec, w_spec,
        ],
        out_specs=row_spec,
        out_shape=jax.ShapeDtypeStruct((N, H), jnp.float32),
    )(p, a, b, deg, wn, we)


def kernel(x, edge_index, edge_attr, W_self, b_self, W_nei, W_edge):
    N, H = x.shape
    E = edge_index.shape[1]
    src = edge_index[0].astype(jnp.int32)
    dst = edge_index[1].astype(jnp.int32)
    ND = ((N + 1279) // 1280) * 1280
    zrows = jnp.zeros((N // 10, H), jnp.float32)
    zdeg = jnp.zeros((ND // 10,), jnp.float32)
    p = _tc_self(N, H, x, W_self, b_self.reshape(1, H))
    a_sum, b_sum, deg = _sc_segment_sums(N, H, E)(
        x, src, dst, edge_attr, zrows, zdeg)
    return _tc_combine(N, H, p, a_sum, b_sum, deg[:N].reshape(N, 1),
                       W_nei, W_edge)


# speedup vs baseline: 1.0078x; 1.0078x over previous
import functools

import jax
import jax.numpy as jnp
from jax import lax
from jax.experimental import pallas as pl
from jax.experimental.pallas import tpu as pltpu
from jax.experimental.pallas import tpu_sc as plsc

_NS = 16
_C = 80


def _sc_segment_sums(N, H, E):
    EPT = E // _NS
    CHUNKS = EPT // _C
    RPT = N // 10
    ND = ((N + 1279) // 1280) * 1280
    DCH = ND // 10
    assert EPT * _NS == E and CHUNKS * _C == EPT and RPT * 10 == N
    assert RPT % 8 == 0 and DCH % 128 == 0 and EPT % 8 == 0 and _C % 8 == 0

    mesh = plsc.VectorSubcoreMesh(core_axis_name="c", subcore_axis_name="s")

    NB = 4
    GROUPS = CHUNKS // NB
    TAIL = CHUNKS - GROUPS * NB

    def body(x_hbm, src_hbm, dst_hbm, ea_hbm, zrows_hbm, zdeg_hbm,
             a_hbm, b_hbm, deg_hbm,
             acc, degacc, idx_s, idx_d, rows_v, ones_v,
             isem, gsem, ssem):
        c = lax.axis_index("c")
        s = lax.axis_index("s")

        @pl.when(s < 10)
        def _zero():
            pltpu.sync_copy(zrows_hbm, acc.at[pl.ds(s * RPT, RPT), :])
            pltpu.sync_copy(zdeg_hbm, degacc.at[pl.ds(s * DCH, DCH)])

        for k in range(_C // 16):
            ones_v[pl.ds(k * 16, 16)] = jnp.full((16,), 1.0, jnp.float32)

        plsc.subcore_barrier()

        tbase = s * EPT
        HALF = NB // 2
        HA = list(range(HALF))
        HB = list(range(HALF, NB))

        def _fire0(slots, goff):
            for i, b in enumerate(slots):
                off = pl.multiple_of(tbase + goff + i * _C, 8)
                pltpu.async_copy(src_hbm.at[pl.ds(off, _C)], idx_s.at[b],
                                 isem.at[b])
                pltpu.async_copy(dst_hbm.at[pl.ds(off, _C)], idx_d.at[b],
                                 isem.at[b])

        def _fire1(slots, goff):
            for i, b in enumerate(slots):
                off = pl.multiple_of(tbase + goff + i * _C, 8)
                pltpu.async_copy(dst_hbm.at[pl.ds(off, _C)], idx_d.at[b],
                                 isem.at[b])
                pltpu.async_copy(ea_hbm.at[pl.ds(off, _C), :], rows_v.at[b],
                                 gsem.at[b])

        def _wait_scat(slots, with_deg):
            for b in slots:
                pltpu.make_async_copy(rows_v.at[b], acc.at[idx_d.at[b]],
                                      ssem.at[b]).wait()
                if with_deg:
                    pltpu.make_async_copy(ones_v, degacc.at[idx_d.at[b]],
                                          ssem.at[b]).wait()

        def _proc1(slots):
            for b in slots:
                pltpu.make_async_copy(dst_hbm.at[pl.ds(0, _C)],
                                      idx_d.at[b], isem.at[b]).wait()
                pltpu.make_async_copy(ea_hbm.at[pl.ds(0, _C), :],
                                      rows_v.at[b], gsem.at[b]).wait()
                pltpu.async_copy(rows_v.at[b], acc.at[idx_d.at[b]],
                                 ssem.at[b], add=True)
                pltpu.async_copy(ones_v, degacc.at[idx_d.at[b]], ssem.at[b],
                                 add=True)

        def _pipeline(fire, proc, with_deg):
            fire(HA, 0)

            def group(g, carry):
                base = g * (NB * _C)

                @pl.when(g > 0)
                def _():
                    _wait_scat(HB, with_deg)
                fire(HB, base + HALF * _C)
                proc(HA)
                _wait_scat(HA, with_deg)

                @pl.when(g + 1 < GROUPS)
                def _():
                    fire(HA, base + NB * _C)
                proc(HB)
                return carry
            lax.fori_loop(0, GROUPS, group, 0)
            _wait_scat(HB, with_deg)

        def _pipeline0():
            _fire0(HA, 0)

            def group(g, carry):
                base = g * (NB * _C)

                @pl.when(g > 0)
                def _():
                    _wait_scat(HB, False)
                _fire0(HB, base + HALF * _C)
                gd = {}
                for b in HA + HB:
                    pltpu.make_async_copy(src_hbm.at[pl.ds(0, _C)],
                                          idx_s.at[b], isem.at[b]).wait()
                    pltpu.make_async_copy(dst_hbm.at[pl.ds(0, _C)],
                                          idx_d.at[b], isem.at[b]).wait()
                    gd[b] = pltpu.async_copy(x_hbm.at[idx_s.at[b]],
                                             rows_v.at[b], gsem.at[b])
                for b in HA:
                    gd[b].wait()
                    pltpu.async_copy(rows_v.at[b], acc.at[idx_d.at[b]],
                                     ssem.at[b], add=True)
                _wait_scat(HA, False)

                @pl.when(g + 1 < GROUPS)
                def _():
                    _fire0(HA, base + NB * _C)
                for b in HB:
                    gd[b].wait()
                    pltpu.async_copy(rows_v.at[b], acc.at[idx_d.at[b]],
                                     ssem.at[b], add=True)
                return carry
            lax.fori_loop(0, GROUPS, group, 0)
            _wait_scat(HB, False)

        @pl.when(c == 0)
        def _core0():
            _pipeline0()
            for t in range(TAIL):
                off = pl.multiple_of(tbase + (GROUPS * NB + t) * _C, 8)
                pltpu.sync_copy(src_hbm.at[pl.ds(off, _C)], idx_s.at[0])
                pltpu.sync_copy(dst_hbm.at[pl.ds(off, _C)], idx_d.at[0])
                pltpu.async_copy(x_hbm.at[idx_s.at[0]], rows_v.at[0],
                                 gsem.at[0]).wait()
                pltpu.sync_copy(rows_v.at[0], acc.at[idx_d.at[0]], add=True)

        @pl.when(c == 1)
        def _core1():
            _pipeline(_fire1, _proc1, True)
            for t in range(TAIL):
                off = pl.multiple_of(tbase + (GROUPS * NB + t) * _C, 8)
                pltpu.sync_copy(dst_hbm.at[pl.ds(off, _C)], idx_d.at[0])
                pltpu.sync_copy(ea_hbm.at[pl.ds(off, _C), :], rows_v.at[0])
                pltpu.sync_copy(rows_v.at[0], acc.at[idx_d.at[0]], add=True)
                pltpu.sync_copy(ones_v, degacc.at[idx_d.at[0]], add=True)

        plsc.subcore_barrier()

        @pl.when(jnp.logical_and(c == 0, s < 10))
        def _out0():
            pltpu.sync_copy(acc.at[pl.ds(s * RPT, RPT), :],
                            a_hbm.at[pl.ds(s * RPT, RPT), :])

        @pl.when(jnp.logical_and(c == 1, s < 10))
        def _out1():
            pltpu.sync_copy(acc.at[pl.ds(s * RPT, RPT), :],
                            b_hbm.at[pl.ds(s * RPT, RPT), :])
            pltpu.sync_copy(degacc.at[pl.ds(s * DCH, DCH)],
                            deg_hbm.at[pl.ds(s * DCH, DCH)])

    return pl.kernel(
        body,
        out_type=(jax.ShapeDtypeStruct((N, H), jnp.float32),
                  jax.ShapeDtypeStruct((N, H), jnp.float32),
                  jax.ShapeDtypeStruct((ND,), jnp.float32)),
        mesh=mesh,
        scratch_types=[
            pltpu.VMEM_SHARED((N, H), jnp.float32),
            pltpu.VMEM_SHARED((ND,), jnp.float32),
            pltpu.VMEM((NB, _C), jnp.int32),
            pltpu.VMEM((NB, _C), jnp.int32),
            pltpu.VMEM((NB, _C, H), jnp.float32),
            pltpu.VMEM((_C,), jnp.float32),
            pltpu.SemaphoreType.DMA((NB,)),
            pltpu.SemaphoreType.DMA((NB,)),
            pltpu.SemaphoreType.DMA((NB,)),
        ],
    )


def _dot_nt(lhs, w):
    return lax.dot_general(lhs, w, (((1,), (1,)), ((), ())),
                           preferred_element_type=jnp.float32)


def _tc_body(x_ref, a_ref, b_ref, deg_ref, ws_ref, wn_ref, we_ref,
             bias_ref, o_ref):
    agg = _dot_nt(a_ref[...], wn_ref[...]) + _dot_nt(b_ref[...], we_ref[...])
    agg = agg / jnp.maximum(deg_ref[...], 1.0)
    y = _dot_nt(x_ref[...], ws_ref[...]) + bias_ref[...] + agg
    o_ref[...] = jnp.maximum(y, 0.0)


def _tc_combine(N, H, x, a, b, deg, wst, wnt, wet, bias):
    BLK = 2000
    assert N % BLK == 0
    grid = (N // BLK,)
    row_spec = pl.BlockSpec((BLK, H), lambda i: (i, 0))
    w_spec = pl.BlockSpec((H, H), lambda i: (0, 0))
    return pl.pallas_call(
        _tc_body,
        grid=grid,
        in_specs=[
            row_spec,
            row_spec,
            row_spec,
            pl.BlockSpec((BLK, 1), lambda i: (i, 0)),
            w_spec, w_spec, w_spec,
            pl.BlockSpec((1, H), lambda i: (0, 0)),
        ],
        out_specs=row_spec,
        out_shape=jax.ShapeDtypeStruct((N, H), jnp.float32),
    )(x, a, b, deg, wst, wnt, wet, bias)


def kernel(x, edge_index, edge_attr, W_self, b_self, W_nei, W_edge):
    N, H = x.shape
    E = edge_index.shape[1]
    src = edge_index[0].astype(jnp.int32)
    dst = edge_index[1].astype(jnp.int32)
    ND = ((N + 1279) // 1280) * 1280
    zrows = jnp.zeros((N // 10, H), jnp.float32)
    zdeg = jnp.zeros((ND // 10,), jnp.float32)
    a_sum, b_sum, deg = _sc_segment_sums(N, H, E)(
        x, src, dst, edge_attr, zrows, zdeg)
    return _tc_combine(N, H, x, a_sum, b_sum, deg[:N].reshape(N, 1),
                       W_self, W_nei, W_edge, b_self.reshape(1, H))
